# baseline (speedup 1.0000x reference)
import jax
import jax.numpy as jnp
from jax.experimental import pallas as pl
from jax.experimental.pallas import tpu as pltpu

N = 10000
EPS = 1e-5
BM = 400


def _body(wt_ref, b_ref, g_ref, be_ref, f_hbm, a_ref, out_ref,
          t_ref, s_ref, q_ref, sem_f):
    i = pl.program_id(0)
    nsteps = pl.num_programs(0)

    @pl.when(i == 0)
    def _():
        f_cp = pltpu.make_async_copy(f_hbm, out_ref, sem_f)
        f_cp.start()
        f_cp.wait()
        t_ref[...] = (
            jnp.dot(out_ref[...], wt_ref[...], preferred_element_type=jnp.float32)
            + b_ref[...]
        )
        s_ref[...] = jnp.zeros_like(s_ref)
        q_ref[...] = jnp.zeros_like(q_ref)

    @pl.when(i > 0)
    def _():
        c = i - 1
        o = jnp.dot(a_ref[...], t_ref[...], preferred_element_type=jnp.float32)
        out_ref[pl.ds(c * BM, BM), :] = o
        s_ref[...] += jnp.sum(o, axis=0, keepdims=True)
        q_ref[...] += jnp.sum(o * o, axis=0, keepdims=True)

    @pl.when(i == nsteps - 1)
    def _():
        mean = s_ref[...] / N
        var = q_ref[...] / N - mean * mean
        inv = jax.lax.rsqrt(var + EPS) * g_ref[...]
        out_ref[...] = jnp.maximum((out_ref[...] - mean) * inv + be_ref[...], 0.0)


def kernel(features, adjacency_matrix, W, b, gamma, beta):
    n, d_in = features.shape
    d_out = W.shape[0]
    grid = n // BM + 1

    return pl.pallas_call(
        _body,
        grid=(grid,),
        in_specs=[
            pl.BlockSpec((d_in, d_out), lambda i: (0, 0)),
            pl.BlockSpec((1, d_out), lambda i: (0, 0)),
            pl.BlockSpec((1, d_out), lambda i: (0, 0)),
            pl.BlockSpec((1, d_out), lambda i: (0, 0)),
            pl.BlockSpec(memory_space=pltpu.MemorySpace.HBM),
            pl.BlockSpec((BM, n), lambda i: (jnp.maximum(i - 1, 0), 0)),
        ],
        out_specs=pl.BlockSpec((n, d_out), lambda i: (0, 0)),
        out_shape=jax.ShapeDtypeStruct((n, d_out), jnp.float32),
        scratch_shapes=[
            pltpu.VMEM((n, d_out), jnp.float32),
            pltpu.VMEM((1, d_out), jnp.float32),
            pltpu.VMEM((1, d_out), jnp.float32),
            pltpu.SemaphoreType.DMA,
        ],
    )(
        W.T,
        b.reshape(1, d_out),
        gamma.reshape(1, d_out),
        beta.reshape(1, d_out),
        features,
        adjacency_matrix,
    )


# speedup vs baseline: 1.0268x; 1.0268x over previous
import jax
import jax.numpy as jnp
from jax.experimental import pallas as pl
from jax.experimental.pallas import tpu as pltpu

N = 10000
EPS = 1e-5
BM = 400


def _body(f_ref, wt_ref, b_ref, g_ref, be_ref, a_ref, out_ref, t_ref, s_ref, q_ref):
    i = pl.program_id(0)
    nsteps = pl.num_programs(0)

    @pl.when(i == 0)
    def _():
        t_ref[...] = (
            jnp.dot(f_ref[...], wt_ref[...], preferred_element_type=jnp.float32)
            + b_ref[...]
        )

    o = jnp.dot(a_ref[...], t_ref[...], preferred_element_type=jnp.float32)
    out_ref[pl.ds(i * BM, BM), :] = o
    ps = jnp.sum(o, axis=0, keepdims=True)
    pq = jnp.sum(o * o, axis=0, keepdims=True)

    @pl.when(i == 0)
    def _():
        s_ref[...] = ps
        q_ref[...] = pq

    @pl.when(i > 0)
    def _():
        s_ref[...] += ps
        q_ref[...] += pq

    @pl.when(i == nsteps - 1)
    def _():
        mean = s_ref[...] / N
        var = q_ref[...] / N - mean * mean
        inv = jax.lax.rsqrt(var + EPS) * g_ref[...]
        out_ref[...] = jnp.maximum((out_ref[...] - mean) * inv + be_ref[...], 0.0)


def kernel(features, adjacency_matrix, W, b, gamma, beta):
    n, d_in = features.shape
    d_out = W.shape[0]
    grid = n // BM

    return pl.pallas_call(
        _body,
        grid=(grid,),
        in_specs=[
            pl.BlockSpec((n, d_in), lambda i: (0, 0)),
            pl.BlockSpec((d_in, d_out), lambda i: (0, 0)),
            pl.BlockSpec((1, d_out), lambda i: (0, 0)),
            pl.BlockSpec((1, d_out), lambda i: (0, 0)),
            pl.BlockSpec((1, d_out), lambda i: (0, 0)),
            pl.BlockSpec((BM, n), lambda i: (i, 0)),
        ],
        out_specs=pl.BlockSpec((n, d_out), lambda i: (0, 0)),
        out_shape=jax.ShapeDtypeStruct((n, d_out), jnp.float32),
        scratch_shapes=[
            pltpu.VMEM((n, d_out), jnp.float32),
            pltpu.VMEM((1, d_out), jnp.float32),
            pltpu.VMEM((1, d_out), jnp.float32),
        ],
    )(
        features,
        W.T,
        b.reshape(1, d_out),
        gamma.reshape(1, d_out),
        beta.reshape(1, d_out),
        adjacency_matrix,
    )
